# Initial kernel scaffold; baseline (speedup 1.0000x reference)
import jax
import jax.numpy as jnp
from jax.experimental import pallas as pl


def kernel(x, edge_index, edge_attr, batch, emb, W1, asrc1, adst1, b1, W2, asrc2, adst2, b2, W3, asrc3, adst3, b3, fc_w, fc_b, wz_w, wz_b, wr_w, wr_b, wh_w, wh_b):
    raise NotImplementedError("write your pallas kernel here")


# speedup vs baseline: 28.7201x; 28.7201x over previous
import functools

import jax
import jax.numpy as jnp
from jax import lax
from jax.experimental import pallas as pl
from jax.experimental.pallas import tpu as pltpu
from jax.experimental.pallas import tpu_sc as plsc

N = 8192
E = 16384
G = 16
HID = 256
AUG = 384
EMB = 64
OUT = 128
BINS_R = 2048 + 16
CH = 128

_mesh = plsc.VectorSubcoreMesh(core_axis_name="c", subcore_axis_name="s")
_SC_PARAMS = pltpu.CompilerParams(needs_layout_passes=False)


def _leaky(x, a):
    return jnp.maximum(x, a * x)


def _gelu(x):
    return x * 0.5 * (1.0 + lax.erf(x * 0.7071067811865476))


@functools.partial(
    pl.kernel,
    mesh=_mesh,
    compiler_params=_SC_PARAMS,
    out_type=(
        jax.ShapeDtypeStruct((N, 128), jnp.float32),
        jax.ShapeDtypeStruct((E,), jnp.int32),
    ),
    scratch_types=[
        pltpu.VMEM((N // 32,), jnp.int32),
        pltpu.VMEM((N // 32, 128), jnp.float32),
        pltpu.VMEM((N,), jnp.int32),
        pltpu.VMEM((E // 32,), jnp.int32),
        pltpu.VMEM((E // 32,), jnp.int32),
        pltpu.SemaphoreType.DMA,
    ],
)
def _s1(nidx_hbm, src_hbm, batch_hbm, emb_hbm, embr_out, bsrc_out,
        idx_v, rows_v, batch_v, srcv, outv, sem):
    c = lax.axis_index("c")
    s = lax.axis_index("s")
    tid = s * 2 + c
    nb = N // 32
    base = tid * nb
    pltpu.sync_copy(nidx_hbm.at[pl.ds(base, nb)], idx_v)
    pltpu.async_copy(emb_hbm.at[idx_v], rows_v, sem).wait()
    pltpu.sync_copy(rows_v, embr_out.at[pl.ds(base, nb)])

    eb = E // 32
    ebase = tid * eb
    pltpu.sync_copy(batch_hbm, batch_v)
    pltpu.sync_copy(src_hbm.at[pl.ds(ebase, eb)], srcv)

    def body(g, carry):
        sidx = srcv[pl.ds(g * 16, 16)]
        outv[pl.ds(g * 16, 16)] = plsc.load_gather(batch_v, [sidx])
        return carry

    lax.fori_loop(0, eb // 16, body, 0)
    pltpu.sync_copy(outv, bsrc_out.at[pl.ds(ebase, eb)])


@functools.partial(
    pl.kernel,
    mesh=_mesh,
    compiler_params=_SC_PARAMS,
    out_type=jax.ShapeDtypeStruct((N, AUG), jnp.float32),
    scratch_types=[
        pltpu.VMEM((N,), jnp.float32),
        pltpu.VMEM((N,), jnp.float32),
        pltpu.VMEM((8, 128), jnp.float32),
        pltpu.VMEM((128,), jnp.int32),
        pltpu.VMEM((16, 128), jnp.int32),
        pltpu.VMEM((16, AUG), jnp.float32),
        pltpu.VMEM((144, AUG), jnp.float32),
        pltpu.SemaphoreType.DMA,
    ],
)
def _bk(hwa_hbm, es_hbm, ed_hbm, m_hbm, pairs_hbm, boff_hbm, out_hbm,
        es_v, ed_v, m_v, boff_v, pbuf, rowb, bins, sem):
    c = lax.axis_index("c")
    s = lax.axis_index("s")
    tid = s * 2 + c
    pltpu.sync_copy(es_hbm, es_v)
    pltpu.sync_copy(ed_hbm, ed_v)
    pltpu.sync_copy(m_hbm, m_v)
    pltpu.sync_copy(boff_hbm, boff_v)
    mrow = m_v[0, pl.ds(0, 16)]
    zm = mrow[0] + mrow[1]
    mshift = jnp.maximum(zm, 0.2 * zm)

    zv = jnp.zeros((16,), jnp.float32)
    for r in range(16):
        for v in range(AUG // 16):
            rowb[r, pl.ds(v * 16, 16)] = zv
    iot = lax.iota(jnp.int32, 16)
    zer16 = iot * 0
    one16 = zer16 + 1

    for p in range(2):
        b = tid + p * 32

        def zb(j, carry):
            for v in range(AUG // 16):
                bins[j, pl.ds(v * 16, 16)] = zv
            return carry

        lax.fori_loop(0, 144, zb, 0)

        bvec = jnp.full((16,), b, jnp.int32)
        elo = plsc.load_gather(boff_v, [bvec])[0]
        ehi = plsc.load_gather(boff_v, [bvec + 1])[0]
        alo = (elo // 16) * 16
        ng = (ehi - alo + 15) // 16

        def grp(k, carry):
            gstart = pl.multiple_of(alo + k * 16, 16)
            pltpu.sync_copy(pairs_hbm.at[pl.ds(gstart, 16)], pbuf)
            evalid = ((gstart + iot) >= elo) & ((gstart + iot) < ehi)
            sidx = plsc.load_gather(pbuf, [iot, zer16])
            didx = plsc.load_gather(pbuf, [iot, one16])
            sidx = jnp.where(evalid, sidx, 0)
            didx = jnp.where(evalid, didx, 0)
            a = plsc.load_gather(es_v, [sidx])
            bb = plsc.load_gather(ed_v, [didx])
            z = a + bb
            e = jnp.maximum(z, 0.2 * z)
            exv = jnp.where(evalid, jnp.exp(e - mshift), 0.0)
            pltpu.async_copy(hwa_hbm.at[sidx], rowb, sem).wait()
            loc = didx - b * 128
            okl = evalid & (loc >= 0) & (loc < 128)
            rr_v = jnp.where(okl, loc, 128)
            for r in range(16):
                sr = exv[r]
                rr = rr_v[r]
                for v in range(AUG // 16):
                    d = pl.ds(v * 16, 16)
                    bins[rr, d] = bins[rr, d] + rowb[r, d] * sr
            return carry

        lax.fori_loop(0, ng, grp, 0)

        pltpu.sync_copy(bins.at[pl.ds(0, 128)],
                        out_hbm.at[pl.ds(b * 128, 128)])


@functools.partial(
    pl.kernel,
    mesh=_mesh,
    compiler_params=_SC_PARAMS,
    out_type=jax.ShapeDtypeStruct((E + 16, 128), jnp.int32),
    scratch_types=[
        pltpu.VMEM((E // 32,), jnp.int32),
        pltpu.VMEM((E // 32,), jnp.int32),
        pltpu.VMEM((E // 32,), jnp.int32),
        pltpu.VMEM((16, 128), jnp.int32),
    ],
)
def _s1b(src_hbm, dst_hbm, posd_hbm, pairs_out, srcv, dstv, posv, combo):
    c = lax.axis_index("c")
    s = lax.axis_index("s")
    tid = s * 2 + c
    eb = E // 32
    ebase = tid * eb
    pltpu.sync_copy(src_hbm.at[pl.ds(ebase, eb)], srcv)
    pltpu.sync_copy(dst_hbm.at[pl.ds(ebase, eb)], dstv)
    pltpu.sync_copy(posd_hbm.at[pl.ds(ebase, eb)], posv)
    iot = lax.iota(jnp.int32, 16)
    zi = iot * 0
    for r in range(16):
        for v in range(8):
            combo[r, pl.ds(v * 16, 16)] = zi

    def body(g, carry):
        gb = g * 16
        sidx = srcv[pl.ds(gb, 16)]
        didx = dstv[pl.ds(gb, 16)]
        pidx = posv[pl.ds(gb, 16)]
        for r in range(16):
            combo[r, pl.ds(0, 16)] = jnp.where(
                iot == 0, sidx[r], jnp.where(iot == 1, didx[r], 0))
        pltpu.sync_copy(combo, pairs_out.at[pidx])
        return carry

    lax.fori_loop(0, eb // 16, body, 0)


@functools.partial(
    pl.kernel,
    mesh=_mesh,
    compiler_params=_SC_PARAMS,
    out_type=jax.ShapeDtypeStruct((E, HID), jnp.float32),
    scratch_types=[
        pltpu.VMEM((E // 32,), jnp.int32),
        pltpu.VMEM((E // 32,), jnp.int32),
        pltpu.VMEM((E // 32,), jnp.int32),
        pltpu.VMEM((16, HID), jnp.float32),
        pltpu.VMEM((16, HID), jnp.float32),
        pltpu.SemaphoreType.DMA,
    ],
)
def _s2(p1_hbm, p2_hbm, src_hbm, dst_hbm, pos_hbm, q_out,
        srcv, dstv, posv, bufa, bufb, sem):
    c = lax.axis_index("c")
    s = lax.axis_index("s")
    tid = s * 2 + c
    eb = E // 32
    ebase = tid * eb
    pltpu.sync_copy(src_hbm.at[pl.ds(ebase, eb)], srcv)
    pltpu.sync_copy(dst_hbm.at[pl.ds(ebase, eb)], dstv)
    pltpu.sync_copy(pos_hbm.at[pl.ds(ebase, eb)], posv)

    def body(g, carry):
        gb = g * 16
        sidx = srcv[pl.ds(gb, 16)]
        didx = dstv[pl.ds(gb, 16)]
        pidx = posv[pl.ds(gb, 16)]
        pltpu.async_copy(p1_hbm.at[sidx], bufa, sem).wait()
        pltpu.async_copy(p2_hbm.at[didx], bufb, sem).wait()
        for r in range(16):
            for v in range(HID // 16):
                d = pl.ds(v * 16, 16)
                bufa[r, d] = bufa[r, d] + bufb[r, d]
        pltpu.sync_copy(bufa, q_out.at[pidx])
        return carry

    lax.fori_loop(0, eb // 16, body, 0)


@functools.partial(
    pl.kernel,
    mesh=_mesh,
    compiler_params=_SC_PARAMS,
    out_type=jax.ShapeDtypeStruct((E * G, 3 * OUT), jnp.float32),
    scratch_types=[
        pltpu.VMEM((16,), jnp.int32),
        pltpu.VMEM((16, 3 * OUT), jnp.float32),
    ],
)
def _s3(a_hbm, off_hbm, apad_out, offv, rowb):
    c = lax.axis_index("c")
    s = lax.axis_index("s")
    tid = s * 2 + c
    eb = E // 32
    tbase = tid * eb
    pltpu.sync_copy(off_hbm, offv)
    offvec = offv[pl.ds(0, 16)]
    offsc = [offvec[j] for j in range(16)]

    def body(g, carry):
        base = tbase + g * 16
        ev = base + lax.iota(jnp.int32, 16)
        cnt = jnp.zeros((16,), jnp.int32)
        for j in range(16):
            cnt = cnt + (ev >= offsc[j]).astype(jnp.int32)
        gv = cnt - 1
        ofg = plsc.load_gather(offv, [gv])
        slot = (ev - ofg) * G + gv
        pltpu.sync_copy(a_hbm.at[pl.ds(base, 16)], rowb)
        pltpu.sync_copy(rowb, apad_out.at[slot])
        return carry

    lax.fori_loop(0, eb // 16, body, 0)


TB = 2048


def _onehot_cum(key2, nb):
    lanes = lax.broadcasted_iota(jnp.int32, (TB, nb), 1)
    oh = (key2 == lanes).astype(jnp.float32)
    cum = oh
    k = 1
    while k < TB:
        sh = jnp.concatenate(
            [jnp.zeros((k, nb), jnp.float32), cum[: TB - k]], axis=0)
        cum = cum + sh
        k *= 2
    return oh, cum


def _excl_lanes(x):
    L = x.shape[1]
    incl = x
    k = 1
    while k < L:
        incl = incl + jnp.concatenate(
            [jnp.zeros((1, k), jnp.float32), incl[:, : L - k]], axis=1)
        k *= 2
    return incl - x


def _t1a_body(bs_ref, dst_ref, cnt16_ref, cnt64_ref):
    i = pl.program_id(0)
    oh16, _ = _onehot_cum(bs_ref[...], G)
    bkt = lax.shift_right_logical(dst_ref[...], 7)
    oh64, _ = _onehot_cum(bkt, 64)
    s16 = jnp.sum(oh16, axis=0, keepdims=True)
    s64 = jnp.sum(oh64, axis=0, keepdims=True)
    s64 = jnp.concatenate([s64, jnp.zeros((1, 64), jnp.float32)], axis=1)

    @pl.when(i == 0)
    def _():
        cnt16_ref[...] = s16
        cnt64_ref[...] = s64

    @pl.when(i > 0)
    def _():
        cnt16_ref[...] = cnt16_ref[...] + s16
        cnt64_ref[...] = cnt64_ref[...] + s64


def _t1b_body(bs_ref, dst_ref, cnt16_ref, cnt64_ref, pos_ref, posd_ref,
              cnt_ref, off_ref, boff_ref, c16_s, c64_s):
    i = pl.program_id(0)

    @pl.when(i == 0)
    def _():
        c16_s[...] = jnp.zeros((1, G), jnp.float32)
        c64_s[...] = jnp.zeros((1, 128), jnp.float32)

    offs16 = _excl_lanes(cnt16_ref[...])
    offs64 = _excl_lanes(cnt64_ref[...])

    oh16, cum16 = _onehot_cum(bs_ref[...], G)
    pos = (jnp.sum(cum16 * oh16, axis=1, keepdims=True) - 1.0
           + jnp.sum(oh16 * (offs16 + c16_s[...]), axis=1, keepdims=True))
    pos_ref[...] = pos.astype(jnp.int32)
    c16_s[...] = c16_s[...] + cum16[TB - 1: TB, :]

    bkt = lax.shift_right_logical(dst_ref[...], 7)
    oh64, cum64 = _onehot_cum(bkt, 64)
    posd = (jnp.sum(cum64 * oh64, axis=1, keepdims=True) - 1.0
            + jnp.sum(oh64 * (offs64[:, :64] + c64_s[...][:, :64]),
                      axis=1, keepdims=True))
    posd_ref[...] = posd.astype(jnp.int32)
    c64_s[...] = c64_s[...] + jnp.concatenate(
        [cum64[TB - 1: TB, :], jnp.zeros((1, 64), jnp.float32)], axis=1)

    cnt_ref[...] = cnt16_ref[...].astype(jnp.int32)
    off_ref[...] = offs16.astype(jnp.int32)
    boff_ref[...] = offs64.astype(jnp.int32)


def _t1(bs2, dst2):
    cnt16, cnt64 = pl.pallas_call(
        _t1a_body,
        grid=(E // TB,),
        in_specs=[
            pl.BlockSpec((TB, 1), lambda i: (i, 0)),
            pl.BlockSpec((TB, 1), lambda i: (i, 0)),
        ],
        out_specs=[
            pl.BlockSpec((1, G), lambda i: (0, 0)),
            pl.BlockSpec((1, 128), lambda i: (0, 0)),
        ],
        out_shape=(
            jax.ShapeDtypeStruct((1, G), jnp.float32),
            jax.ShapeDtypeStruct((1, 128), jnp.float32),
        ),
    )(bs2, dst2)
    return pl.pallas_call(
        _t1b_body,
        grid=(E // TB,),
        in_specs=[
            pl.BlockSpec((TB, 1), lambda i: (i, 0)),
            pl.BlockSpec((TB, 1), lambda i: (i, 0)),
            pl.BlockSpec((1, G), lambda i: (0, 0)),
            pl.BlockSpec((1, 128), lambda i: (0, 0)),
        ],
        out_specs=[
            pl.BlockSpec((TB, 1), lambda i: (i, 0)),
            pl.BlockSpec((TB, 1), lambda i: (i, 0)),
            pl.BlockSpec((1, G), lambda i: (0, 0)),
            pl.BlockSpec((1, G), lambda i: (0, 0)),
            pl.BlockSpec((1, 128), lambda i: (0, 0)),
        ],
        out_shape=(
            jax.ShapeDtypeStruct((E, 1), jnp.int32),
            jax.ShapeDtypeStruct((E, 1), jnp.int32),
            jax.ShapeDtypeStruct((1, G), jnp.int32),
            jax.ShapeDtypeStruct((1, G), jnp.int32),
            jax.ShapeDtypeStruct((1, 128), jnp.int32),
        ),
        scratch_shapes=[pltpu.VMEM((1, G), jnp.float32),
                        pltpu.VMEM((1, 128), jnp.float32)],
    )(bs2, dst2, cnt16, cnt64)


def _aug_and_logits(i, hw, as_v, ad_v, hwa_ref, es_ref, ed_ref, m_ref):
    blk = hw.shape[0]
    hwa_ref[...] = jnp.concatenate(
        [hw, jnp.ones((blk, 1), jnp.float32), jnp.zeros((blk, 127), jnp.float32)],
        axis=1)
    es = jnp.dot(hw, as_v, preferred_element_type=jnp.float32)
    ed = jnp.dot(hw, ad_v, preferred_element_type=jnp.float32)
    es_ref[...] = es
    ed_ref[...] = ed
    ce = jnp.max(es)
    cd = jnp.max(ed)
    ri = lax.broadcasted_iota(jnp.int32, (8, 128), 0)
    ci = lax.broadcasted_iota(jnp.int32, (8, 128), 1)
    row = jnp.where((ri == 0) & (ci == 0), ce,
                    jnp.where((ri == 0) & (ci == 1), cd, -1e30))

    @pl.when(i == 0)
    def _():
        m_ref[...] = row

    @pl.when(i > 0)
    def _():
        m_ref[...] = jnp.maximum(m_ref[...], row)


def _t2a_body(xa_ref, xb_ref, wa_ref, wb_ref, as_ref, ad_ref,
              hwa_ref, es_ref, ed_ref, m_ref):
    i = pl.program_id(0)
    hw = jnp.dot(xa_ref[...], wa_ref[...], preferred_element_type=jnp.float32)
    hw = hw + jnp.dot(xb_ref[...][:, :EMB], wb_ref[...], preferred_element_type=jnp.float32)
    _aug_and_logits(i, hw, as_ref[...], ad_ref[...], hwa_ref, es_ref, ed_ref, m_ref)


def _t2b_body(p_ref, w_ref, bias_ref, as_ref, ad_ref,
              hwa_ref, es_ref, ed_ref, m_ref):
    i = pl.program_id(0)
    p = p_ref[...]
    num = p[:, :HID]
    den = p[:, HID:HID + 1]
    h = _gelu(num / (den + 1e-16) + bias_ref[...])
    hw = jnp.dot(h, w_ref[...], preferred_element_type=jnp.float32)
    _aug_and_logits(i, hw, as_ref[...], ad_ref[...], hwa_ref, es_ref, ed_ref, m_ref)


_T2_OUT = (
    jax.ShapeDtypeStruct((N, AUG), jnp.float32),
    jax.ShapeDtypeStruct((N, 1), jnp.float32),
    jax.ShapeDtypeStruct((N, 1), jnp.float32),
    jax.ShapeDtypeStruct((8, 128), jnp.float32),
)
_T2_OUT_SPECS = [
    pl.BlockSpec((1024, AUG), lambda i: (i, 0)),
    pl.BlockSpec((1024, 1), lambda i: (i, 0)),
    pl.BlockSpec((1024, 1), lambda i: (i, 0)),
    pl.BlockSpec((8, 128), lambda i: (0, 0)),
]


def _t2a(x256, embr, wa, wb, as2, ad2):
    return pl.pallas_call(
        _t2a_body,
        grid=(N // 1024,),
        in_specs=[
            pl.BlockSpec((1024, HID), lambda i: (i, 0)),
            pl.BlockSpec((1024, 128), lambda i: (i, 0)),
            pl.BlockSpec((HID, HID), lambda i: (0, 0)),
            pl.BlockSpec((EMB, HID), lambda i: (0, 0)),
            pl.BlockSpec((HID, 1), lambda i: (0, 0)),
            pl.BlockSpec((HID, 1), lambda i: (0, 0)),
        ],
        out_specs=_T2_OUT_SPECS,
        out_shape=_T2_OUT,
    )(x256, embr, wa, wb, as2, ad2)


def _t2b(numa, w, bias, as2, ad2):
    return pl.pallas_call(
        _t2b_body,
        grid=(N // 1024,),
        in_specs=[
            pl.BlockSpec((1024, AUG), lambda i: (i, 0)),
            pl.BlockSpec((HID, HID), lambda i: (0, 0)),
            pl.BlockSpec((1, HID), lambda i: (0, 0)),
            pl.BlockSpec((HID, 1), lambda i: (0, 0)),
            pl.BlockSpec((HID, 1), lambda i: (0, 0)),
        ],
        out_specs=_T2_OUT_SPECS,
        out_shape=_T2_OUT,
    )(numa, w, bias, as2, ad2)


def _t5_body(p_ref, bias_ref, w1_ref, w2_ref, p1_ref, p2_ref):
    p = p_ref[...]
    num = p[:, :HID]
    den = p[:, HID:HID + 1]
    h = _gelu(num / (den + 1e-16) + bias_ref[...])
    p1_ref[...] = jnp.dot(h, w1_ref[...], preferred_element_type=jnp.float32)
    p2_ref[...] = jnp.dot(h, w2_ref[...], preferred_element_type=jnp.float32)


def _t5(numa, bias, w1, w2):
    return pl.pallas_call(
        _t5_body,
        grid=(N // 1024,),
        in_specs=[
            pl.BlockSpec((1024, AUG), lambda i: (i, 0)),
            pl.BlockSpec((1, HID), lambda i: (0, 0)),
            pl.BlockSpec((HID, HID), lambda i: (0, 0)),
            pl.BlockSpec((HID, HID), lambda i: (0, 0)),
        ],
        out_specs=[
            pl.BlockSpec((1024, HID), lambda i: (i, 0)),
            pl.BlockSpec((1024, HID), lambda i: (i, 0)),
        ],
        out_shape=(
            jax.ShapeDtypeStruct((N, HID), jnp.float32),
            jax.ShapeDtypeStruct((N, HID), jnp.float32),
        ),
    )(numa, bias, w1, w2)


def _t3_body(q_ref, b_ref, w_ref, bb_ref, a_ref):
    ef = _gelu(q_ref[...] + b_ref[...])
    a_ref[...] = jnp.dot(ef, w_ref[...],
                         preferred_element_type=jnp.float32) + bb_ref[...]


def _t3(q, fcb, wgru, bgru):
    return pl.pallas_call(
        _t3_body,
        grid=(E // 2048,),
        in_specs=[
            pl.BlockSpec((2048, HID), lambda i: (i, 0)),
            pl.BlockSpec((1, HID), lambda i: (0, 0)),
            pl.BlockSpec((HID, 3 * OUT), lambda i: (0, 0)),
            pl.BlockSpec((1, 3 * OUT), lambda i: (0, 0)),
        ],
        out_specs=pl.BlockSpec((2048, 3 * OUT), lambda i: (i, 0)),
        out_shape=jax.ShapeDtypeStruct((E, 3 * OUT), jnp.float32),
    )(q, fcb, wgru, bgru)


def _t4_body(cnt_ref, a_ref, wz_ref, wr_ref, wh_ref, out_ref, h_s):
    i = pl.program_id(0)

    @pl.when(i == 0)
    def _():
        h_s[...] = jnp.zeros((G, OUT), jnp.float32)

    cnt = cnt_ref[...]
    cnt2 = cnt.reshape(G, 1)
    maxc = jnp.max(cnt)
    t0 = i * CH

    @pl.when(t0 < maxc)
    def _():
        wz = wz_ref[...]
        wr = wr_ref[...]
        wh = wh_ref[...]

        def body(t, h):
            row = a_ref[t]
            xz = row[:, :OUT]
            xr = row[:, OUT:2 * OUT]
            xh = row[:, 2 * OUT:]
            z = jax.nn.sigmoid(xz + jnp.dot(h, wz, preferred_element_type=jnp.float32))
            r = jax.nn.sigmoid(xr + jnp.dot(h, wr, preferred_element_type=jnp.float32))
            ht = _leaky(xh + jnp.dot(r * h, wh, preferred_element_type=jnp.float32), 0.01)
            nh = (1.0 - z) * h + z * ht
            valid = (t0 + t) < cnt2
            return jnp.where(valid, nh, h)

        h_s[...] = lax.fori_loop(0, CH, body, h_s[...])

    out_ref[...] = h_s[...]


def _t4(cnt, apad, wz, wr, wh):
    return pl.pallas_call(
        _t4_body,
        grid=(E // CH,),
        in_specs=[
            pl.BlockSpec((1, G), lambda i: (0, 0)),
            pl.BlockSpec((CH, G, 3 * OUT), lambda i: (i, 0, 0)),
            pl.BlockSpec((OUT, OUT), lambda i: (0, 0)),
            pl.BlockSpec((OUT, OUT), lambda i: (0, 0)),
            pl.BlockSpec((OUT, OUT), lambda i: (0, 0)),
        ],
        out_specs=pl.BlockSpec((G, OUT), lambda i: (0, 0)),
        out_shape=jax.ShapeDtypeStruct((G, OUT), jnp.float32),
        scratch_shapes=[pltpu.VMEM((G, OUT), jnp.float32)],
    )(cnt, apad, wz, wr, wh)


def kernel(x, edge_index, edge_attr, batch, emb, W1, asrc1, adst1, b1,
           W2, asrc2, adst2, b2, W3, asrc3, adst3, b3, fc_w, fc_b,
           wz_w, wz_b, wr_w, wr_b, wh_w, wh_b):
    src = edge_index[0]
    dst = edge_index[1]
    nidx = x[:, -1].astype(jnp.int32)
    x256 = x[:, :HID]

    embp = jnp.concatenate([emb, jnp.zeros((emb.shape[0], 128 - EMB), jnp.float32)], axis=1)
    embr, bsrc = _s1(nidx, src, batch, embp)
    pos, posd, cnt, off, boff = _t1(bsrc.reshape(E, 1), dst.reshape(E, 1))
    pos = pos.reshape(E)
    pairs = _s1b(src, dst, posd.reshape(E))
    boff = boff.reshape(128)

    hwa, es, ed, m = _t2a(x256, embr, W1[:HID], W1[HID:],
                          asrc1.reshape(HID, 1), adst1.reshape(HID, 1))
    numa = _bk(hwa, es.reshape(N), ed.reshape(N), m, pairs, boff)

    hwa, es, ed, m = _t2b(numa, W2, b1.reshape(1, HID),
                          asrc2.reshape(HID, 1), adst2.reshape(HID, 1))
    numa = _bk(hwa, es.reshape(N), ed.reshape(N), m, pairs, boff)

    hwa, es, ed, m = _t2b(numa, W3, b2.reshape(1, HID),
                          asrc3.reshape(HID, 1), adst3.reshape(HID, 1))
    numa = _bk(hwa, es.reshape(N), ed.reshape(N), m, pairs, boff)

    p1, p2 = _t5(numa, b3.reshape(1, HID), fc_w[:HID], fc_w[HID:])
    q = _s2(p1, p2, src, dst, pos)

    wgru = jnp.concatenate([wz_w[:HID], wr_w[:HID], wh_w[:HID]], axis=1)
    bgru = jnp.concatenate([wz_b, wr_b, wh_b]).reshape(1, 3 * OUT)
    a = _t3(q, fc_b.reshape(1, HID), wgru, bgru)

    apad = _s3(a, off.reshape(G))
    return _t4(cnt, apad.reshape(E, G, 3 * OUT),
               wz_w[HID:], wr_w[HID:], wh_w[HID:])


# speedup vs baseline: 30.2617x; 1.0537x over previous
import functools

import jax
import jax.numpy as jnp
from jax import lax
from jax.experimental import pallas as pl
from jax.experimental.pallas import tpu as pltpu
from jax.experimental.pallas import tpu_sc as plsc

N = 8192
E = 16384
G = 16
HID = 256
AUG = 384
EMB = 64
OUT = 128
BINS_R = 2048 + 16
CH = 128

_mesh = plsc.VectorSubcoreMesh(core_axis_name="c", subcore_axis_name="s")
_SC_PARAMS = pltpu.CompilerParams(needs_layout_passes=False)


def _leaky(x, a):
    return jnp.maximum(x, a * x)


def _gelu(x):
    return x * 0.5 * (1.0 + lax.erf(x * 0.7071067811865476))


@functools.partial(
    pl.kernel,
    mesh=_mesh,
    compiler_params=_SC_PARAMS,
    out_type=(
        jax.ShapeDtypeStruct((N, 128), jnp.float32),
        jax.ShapeDtypeStruct((E,), jnp.int32),
    ),
    scratch_types=[
        pltpu.VMEM((N // 32,), jnp.int32),
        pltpu.VMEM((N // 32, 128), jnp.float32),
        pltpu.VMEM((N,), jnp.int32),
        pltpu.VMEM((E // 32,), jnp.int32),
        pltpu.VMEM((E // 32,), jnp.int32),
        pltpu.SemaphoreType.DMA,
    ],
)
def _s1(nidx_hbm, src_hbm, batch_hbm, emb_hbm, embr_out, bsrc_out,
        idx_v, rows_v, batch_v, srcv, outv, sem):
    c = lax.axis_index("c")
    s = lax.axis_index("s")
    tid = s * 2 + c
    nb = N // 32
    base = tid * nb
    pltpu.sync_copy(nidx_hbm.at[pl.ds(base, nb)], idx_v)
    pltpu.async_copy(emb_hbm.at[idx_v], rows_v, sem).wait()
    pltpu.sync_copy(rows_v, embr_out.at[pl.ds(base, nb)])

    eb = E // 32
    ebase = tid * eb
    pltpu.sync_copy(batch_hbm, batch_v)
    pltpu.sync_copy(src_hbm.at[pl.ds(ebase, eb)], srcv)

    def body(g, carry):
        sidx = srcv[pl.ds(g * 16, 16)]
        outv[pl.ds(g * 16, 16)] = plsc.load_gather(batch_v, [sidx])
        return carry

    lax.fori_loop(0, eb // 16, body, 0)
    pltpu.sync_copy(outv, bsrc_out.at[pl.ds(ebase, eb)])


@functools.partial(
    pl.kernel,
    mesh=_mesh,
    compiler_params=_SC_PARAMS,
    out_type=jax.ShapeDtypeStruct((N, AUG), jnp.float32),
    scratch_types=[
        pltpu.VMEM((N,), jnp.float32),
        pltpu.VMEM((N,), jnp.float32),
        pltpu.VMEM((8, 128), jnp.float32),
        pltpu.VMEM((128,), jnp.int32),
        pltpu.VMEM((16, 128), jnp.int32),
        pltpu.VMEM((32, AUG), jnp.float32),
        pltpu.VMEM((144, AUG), jnp.float32),
        pltpu.SemaphoreType.DMA,
    ],
)
def _bk(hwa_hbm, es_hbm, ed_hbm, m_hbm, pairs_hbm, boff_hbm, out_hbm,
        es_v, ed_v, m_v, boff_v, pbuf, rowb, bins, sem):
    c = lax.axis_index("c")
    s = lax.axis_index("s")
    tid = s * 2 + c
    pltpu.sync_copy(es_hbm, es_v)
    pltpu.sync_copy(ed_hbm, ed_v)
    pltpu.sync_copy(m_hbm, m_v)
    pltpu.sync_copy(boff_hbm, boff_v)
    mrow = m_v[0, pl.ds(0, 16)]
    zm = mrow[0] + mrow[1]
    mshift = jnp.maximum(zm, 0.2 * zm)

    zv = jnp.zeros((16,), jnp.float32)
    iot = lax.iota(jnp.int32, 16)
    zer16 = iot * 0
    one16 = zer16 + 1

    for p in range(2):
        b = tid + p * 32

        def zb(j, carry):
            for v in range(AUG // 16):
                bins[j, pl.ds(v * 16, 16)] = zv
            return carry

        lax.fori_loop(0, 144, zb, 0)

        bvec = jnp.full((16,), b, jnp.int32)
        elo = plsc.load_gather(boff_v, [bvec])[0]
        ehi = plsc.load_gather(boff_v, [bvec + 1])[0]
        alo = (elo // 16) * 16
        ng = (ehi - alo + 15) // 16

        def fetch(k):
            gstart = pl.multiple_of(alo + k * 16, 16)
            pltpu.sync_copy(pairs_hbm.at[pl.ds(gstart, 16)], pbuf)
            evalid = ((gstart + iot) >= elo) & ((gstart + iot) < ehi)
            sidx = plsc.load_gather(pbuf, [iot, zer16])
            didx = plsc.load_gather(pbuf, [iot, one16])
            sidx = jnp.where(evalid, sidx, 0)
            didx = jnp.where(evalid, didx, 0)
            a = plsc.load_gather(es_v, [sidx])
            bb = plsc.load_gather(ed_v, [didx])
            z = a + bb
            e = jnp.maximum(z, 0.2 * z)
            exv = jnp.where(evalid, jnp.exp(e - mshift), 0.0)
            slot = lax.rem(k, 2) * 16
            cp = pltpu.async_copy(hwa_hbm.at[sidx], rowb.at[pl.ds(slot, 16)],
                                  sem)
            loc = didx - b * 128
            okl = evalid & (loc >= 0) & (loc < 128)
            rr_v = jnp.where(okl, loc, 128)
            return cp, exv, rr_v

        @pl.when(ng > 0)
        def _():
            cp0, ex0, rr0 = fetch(0)

            def grp(k, carry):
                exv, rr_v = carry

                def more():
                    _, exn, rrn = fetch(k + 1)
                    return exn, rrn

                nxt = lax.cond(k + 1 < ng, more, lambda: carry)
                pltpu.make_async_copy(
                    hwa_hbm.at[pl.ds(0, 16)],
                    rowb.at[pl.ds(0, 16)], sem).wait()
                slot = lax.rem(k, 2) * 16
                for r in range(16):
                    sr = exv[r]
                    rr = rr_v[r]
                    for v in range(AUG // 16):
                        d = pl.ds(v * 16, 16)
                        bins[rr, d] = bins[rr, d] + rowb[slot + r, d] * sr
                return nxt

            lax.fori_loop(0, ng, grp, (ex0, rr0))


        pltpu.sync_copy(bins.at[pl.ds(0, 128)],
                        out_hbm.at[pl.ds(b * 128, 128)])


@functools.partial(
    pl.kernel,
    mesh=_mesh,
    compiler_params=_SC_PARAMS,
    out_type=jax.ShapeDtypeStruct((E + 16, 128), jnp.int32),
    scratch_types=[
        pltpu.VMEM((E // 32,), jnp.int32),
        pltpu.VMEM((E // 32,), jnp.int32),
        pltpu.VMEM((E // 32,), jnp.int32),
        pltpu.VMEM((16, 128), jnp.int32),
    ],
)
def _s1b(src_hbm, dst_hbm, posd_hbm, pairs_out, srcv, dstv, posv, combo):
    c = lax.axis_index("c")
    s = lax.axis_index("s")
    tid = s * 2 + c
    eb = E // 32
    ebase = tid * eb
    pltpu.sync_copy(src_hbm.at[pl.ds(ebase, eb)], srcv)
    pltpu.sync_copy(dst_hbm.at[pl.ds(ebase, eb)], dstv)
    pltpu.sync_copy(posd_hbm.at[pl.ds(ebase, eb)], posv)
    iot = lax.iota(jnp.int32, 16)
    zi = iot * 0
    for r in range(16):
        for v in range(8):
            combo[r, pl.ds(v * 16, 16)] = zi

    def body(g, carry):
        gb = g * 16
        sidx = srcv[pl.ds(gb, 16)]
        didx = dstv[pl.ds(gb, 16)]
        pidx = posv[pl.ds(gb, 16)]
        for r in range(16):
            combo[r, pl.ds(0, 16)] = jnp.where(
                iot == 0, sidx[r], jnp.where(iot == 1, didx[r], 0))
        pltpu.sync_copy(combo, pairs_out.at[pidx])
        return carry

    lax.fori_loop(0, eb // 16, body, 0)


@functools.partial(
    pl.kernel,
    mesh=_mesh,
    compiler_params=_SC_PARAMS,
    out_type=jax.ShapeDtypeStruct((E, HID), jnp.float32),
    scratch_types=[
        pltpu.VMEM((E // 32,), jnp.int32),
        pltpu.VMEM((E // 32,), jnp.int32),
        pltpu.VMEM((E // 32,), jnp.int32),
        pltpu.VMEM((16, HID), jnp.float32),
        pltpu.VMEM((16, HID), jnp.float32),
        pltpu.SemaphoreType.DMA,
    ],
)
def _s2(p1_hbm, p2_hbm, src_hbm, dst_hbm, pos_hbm, q_out,
        srcv, dstv, posv, bufa, bufb, sem):
    c = lax.axis_index("c")
    s = lax.axis_index("s")
    tid = s * 2 + c
    eb = E // 32
    ebase = tid * eb
    pltpu.sync_copy(src_hbm.at[pl.ds(ebase, eb)], srcv)
    pltpu.sync_copy(dst_hbm.at[pl.ds(ebase, eb)], dstv)
    pltpu.sync_copy(pos_hbm.at[pl.ds(ebase, eb)], posv)

    def body(g, carry):
        gb = g * 16
        sidx = srcv[pl.ds(gb, 16)]
        didx = dstv[pl.ds(gb, 16)]
        pidx = posv[pl.ds(gb, 16)]
        pltpu.async_copy(p1_hbm.at[sidx], bufa, sem).wait()
        pltpu.async_copy(p2_hbm.at[didx], bufb, sem).wait()
        for r in range(16):
            for v in range(HID // 16):
                d = pl.ds(v * 16, 16)
                bufa[r, d] = bufa[r, d] + bufb[r, d]
        pltpu.sync_copy(bufa, q_out.at[pidx])
        return carry

    lax.fori_loop(0, eb // 16, body, 0)


@functools.partial(
    pl.kernel,
    mesh=_mesh,
    compiler_params=_SC_PARAMS,
    out_type=jax.ShapeDtypeStruct((E * G, 3 * OUT), jnp.float32),
    scratch_types=[
        pltpu.VMEM((16,), jnp.int32),
        pltpu.VMEM((16, 3 * OUT), jnp.float32),
    ],
)
def _s3(a_hbm, off_hbm, apad_out, offv, rowb):
    c = lax.axis_index("c")
    s = lax.axis_index("s")
    tid = s * 2 + c
    eb = E // 32
    tbase = tid * eb
    pltpu.sync_copy(off_hbm, offv)
    offvec = offv[pl.ds(0, 16)]
    offsc = [offvec[j] for j in range(16)]

    def body(g, carry):
        base = tbase + g * 16
        ev = base + lax.iota(jnp.int32, 16)
        cnt = jnp.zeros((16,), jnp.int32)
        for j in range(16):
            cnt = cnt + (ev >= offsc[j]).astype(jnp.int32)
        gv = cnt - 1
        ofg = plsc.load_gather(offv, [gv])
        slot = (ev - ofg) * G + gv
        pltpu.sync_copy(a_hbm.at[pl.ds(base, 16)], rowb)
        pltpu.sync_copy(rowb, apad_out.at[slot])
        return carry

    lax.fori_loop(0, eb // 16, body, 0)


TB = 2048


def _onehot_cum(key2, nb):
    lanes = lax.broadcasted_iota(jnp.int32, (TB, nb), 1)
    oh = (key2 == lanes).astype(jnp.float32)
    cum = oh
    k = 1
    while k < TB:
        sh = jnp.concatenate(
            [jnp.zeros((k, nb), jnp.float32), cum[: TB - k]], axis=0)
        cum = cum + sh
        k *= 2
    return oh, cum


def _excl_lanes(x):
    L = x.shape[1]
    incl = x
    k = 1
    while k < L:
        incl = incl + jnp.concatenate(
            [jnp.zeros((1, k), jnp.float32), incl[:, : L - k]], axis=1)
        k *= 2
    return incl - x


def _t1a_body(bs_ref, dst_ref, cnt16_ref, cnt64_ref):
    i = pl.program_id(0)
    oh16, _ = _onehot_cum(bs_ref[...], G)
    bkt = lax.shift_right_logical(dst_ref[...], 7)
    oh64, _ = _onehot_cum(bkt, 64)
    s16 = jnp.sum(oh16, axis=0, keepdims=True)
    s64 = jnp.sum(oh64, axis=0, keepdims=True)
    s64 = jnp.concatenate([s64, jnp.zeros((1, 64), jnp.float32)], axis=1)

    @pl.when(i == 0)
    def _():
        cnt16_ref[...] = s16
        cnt64_ref[...] = s64

    @pl.when(i > 0)
    def _():
        cnt16_ref[...] = cnt16_ref[...] + s16
        cnt64_ref[...] = cnt64_ref[...] + s64


def _t1b_body(bs_ref, dst_ref, cnt16_ref, cnt64_ref, pos_ref, posd_ref,
              cnt_ref, off_ref, boff_ref, c16_s, c64_s):
    i = pl.program_id(0)

    @pl.when(i == 0)
    def _():
        c16_s[...] = jnp.zeros((1, G), jnp.float32)
        c64_s[...] = jnp.zeros((1, 128), jnp.float32)

    offs16 = _excl_lanes(cnt16_ref[...])
    offs64 = _excl_lanes(cnt64_ref[...])

    oh16, cum16 = _onehot_cum(bs_ref[...], G)
    pos = (jnp.sum(cum16 * oh16, axis=1, keepdims=True) - 1.0
           + jnp.sum(oh16 * (offs16 + c16_s[...]), axis=1, keepdims=True))
    pos_ref[...] = pos.astype(jnp.int32)
    c16_s[...] = c16_s[...] + cum16[TB - 1: TB, :]

    bkt = lax.shift_right_logical(dst_ref[...], 7)
    oh64, cum64 = _onehot_cum(bkt, 64)
    posd = (jnp.sum(cum64 * oh64, axis=1, keepdims=True) - 1.0
            + jnp.sum(oh64 * (offs64[:, :64] + c64_s[...][:, :64]),
                      axis=1, keepdims=True))
    posd_ref[...] = posd.astype(jnp.int32)
    c64_s[...] = c64_s[...] + jnp.concatenate(
        [cum64[TB - 1: TB, :], jnp.zeros((1, 64), jnp.float32)], axis=1)

    cnt_ref[...] = cnt16_ref[...].astype(jnp.int32)
    off_ref[...] = offs16.astype(jnp.int32)
    boff_ref[...] = offs64.astype(jnp.int32)


def _t1(bs2, dst2):
    cnt16, cnt64 = pl.pallas_call(
        _t1a_body,
        grid=(E // TB,),
        in_specs=[
            pl.BlockSpec((TB, 1), lambda i: (i, 0)),
            pl.BlockSpec((TB, 1), lambda i: (i, 0)),
        ],
        out_specs=[
            pl.BlockSpec((1, G), lambda i: (0, 0)),
            pl.BlockSpec((1, 128), lambda i: (0, 0)),
        ],
        out_shape=(
            jax.ShapeDtypeStruct((1, G), jnp.float32),
            jax.ShapeDtypeStruct((1, 128), jnp.float32),
        ),
    )(bs2, dst2)
    return pl.pallas_call(
        _t1b_body,
        grid=(E // TB,),
        in_specs=[
            pl.BlockSpec((TB, 1), lambda i: (i, 0)),
            pl.BlockSpec((TB, 1), lambda i: (i, 0)),
            pl.BlockSpec((1, G), lambda i: (0, 0)),
            pl.BlockSpec((1, 128), lambda i: (0, 0)),
        ],
        out_specs=[
            pl.BlockSpec((TB, 1), lambda i: (i, 0)),
            pl.BlockSpec((TB, 1), lambda i: (i, 0)),
            pl.BlockSpec((1, G), lambda i: (0, 0)),
            pl.BlockSpec((1, G), lambda i: (0, 0)),
            pl.BlockSpec((1, 128), lambda i: (0, 0)),
        ],
        out_shape=(
            jax.ShapeDtypeStruct((E, 1), jnp.int32),
            jax.ShapeDtypeStruct((E, 1), jnp.int32),
            jax.ShapeDtypeStruct((1, G), jnp.int32),
            jax.ShapeDtypeStruct((1, G), jnp.int32),
            jax.ShapeDtypeStruct((1, 128), jnp.int32),
        ),
        scratch_shapes=[pltpu.VMEM((1, G), jnp.float32),
                        pltpu.VMEM((1, 128), jnp.float32)],
    )(bs2, dst2, cnt16, cnt64)


def _aug_and_logits(i, hw, as_v, ad_v, hwa_ref, es_ref, ed_ref, m_ref):
    blk = hw.shape[0]
    hwa_ref[...] = jnp.concatenate(
        [hw, jnp.ones((blk, 1), jnp.float32), jnp.zeros((blk, 127), jnp.float32)],
        axis=1)
    es = jnp.dot(hw, as_v, preferred_element_type=jnp.float32)
    ed = jnp.dot(hw, ad_v, preferred_element_type=jnp.float32)
    es_ref[...] = es
    ed_ref[...] = ed
    ce = jnp.max(es)
    cd = jnp.max(ed)
    ri = lax.broadcasted_iota(jnp.int32, (8, 128), 0)
    ci = lax.broadcasted_iota(jnp.int32, (8, 128), 1)
    row = jnp.where((ri == 0) & (ci == 0), ce,
                    jnp.where((ri == 0) & (ci == 1), cd, -1e30))

    @pl.when(i == 0)
    def _():
        m_ref[...] = row

    @pl.when(i > 0)
    def _():
        m_ref[...] = jnp.maximum(m_ref[...], row)


def _t2a_body(xa_ref, xb_ref, wa_ref, wb_ref, as_ref, ad_ref,
              hwa_ref, es_ref, ed_ref, m_ref):
    i = pl.program_id(0)
    hw = jnp.dot(xa_ref[...], wa_ref[...], preferred_element_type=jnp.float32)
    hw = hw + jnp.dot(xb_ref[...][:, :EMB], wb_ref[...], preferred_element_type=jnp.float32)
    _aug_and_logits(i, hw, as_ref[...], ad_ref[...], hwa_ref, es_ref, ed_ref, m_ref)


def _t2b_body(p_ref, w_ref, bias_ref, as_ref, ad_ref,
              hwa_ref, es_ref, ed_ref, m_ref):
    i = pl.program_id(0)
    p = p_ref[...]
    num = p[:, :HID]
    den = p[:, HID:HID + 1]
    h = _gelu(num / (den + 1e-16) + bias_ref[...])
    hw = jnp.dot(h, w_ref[...], preferred_element_type=jnp.float32)
    _aug_and_logits(i, hw, as_ref[...], ad_ref[...], hwa_ref, es_ref, ed_ref, m_ref)


_T2_OUT = (
    jax.ShapeDtypeStruct((N, AUG), jnp.float32),
    jax.ShapeDtypeStruct((N, 1), jnp.float32),
    jax.ShapeDtypeStruct((N, 1), jnp.float32),
    jax.ShapeDtypeStruct((8, 128), jnp.float32),
)
_T2_OUT_SPECS = [
    pl.BlockSpec((1024, AUG), lambda i: (i, 0)),
    pl.BlockSpec((1024, 1), lambda i: (i, 0)),
    pl.BlockSpec((1024, 1), lambda i: (i, 0)),
    pl.BlockSpec((8, 128), lambda i: (0, 0)),
]


def _t2a(x256, embr, wa, wb, as2, ad2):
    return pl.pallas_call(
        _t2a_body,
        grid=(N // 1024,),
        in_specs=[
            pl.BlockSpec((1024, HID), lambda i: (i, 0)),
            pl.BlockSpec((1024, 128), lambda i: (i, 0)),
            pl.BlockSpec((HID, HID), lambda i: (0, 0)),
            pl.BlockSpec((EMB, HID), lambda i: (0, 0)),
            pl.BlockSpec((HID, 1), lambda i: (0, 0)),
            pl.BlockSpec((HID, 1), lambda i: (0, 0)),
        ],
        out_specs=_T2_OUT_SPECS,
        out_shape=_T2_OUT,
    )(x256, embr, wa, wb, as2, ad2)


def _t2b(numa, w, bias, as2, ad2):
    return pl.pallas_call(
        _t2b_body,
        grid=(N // 1024,),
        in_specs=[
            pl.BlockSpec((1024, AUG), lambda i: (i, 0)),
            pl.BlockSpec((HID, HID), lambda i: (0, 0)),
            pl.BlockSpec((1, HID), lambda i: (0, 0)),
            pl.BlockSpec((HID, 1), lambda i: (0, 0)),
            pl.BlockSpec((HID, 1), lambda i: (0, 0)),
        ],
        out_specs=_T2_OUT_SPECS,
        out_shape=_T2_OUT,
    )(numa, w, bias, as2, ad2)


def _t5_body(p_ref, bias_ref, w1_ref, w2_ref, p1_ref, p2_ref):
    p = p_ref[...]
    num = p[:, :HID]
    den = p[:, HID:HID + 1]
    h = _gelu(num / (den + 1e-16) + bias_ref[...])
    p1_ref[...] = jnp.dot(h, w1_ref[...], preferred_element_type=jnp.float32)
    p2_ref[...] = jnp.dot(h, w2_ref[...], preferred_element_type=jnp.float32)


def _t5(numa, bias, w1, w2):
    return pl.pallas_call(
        _t5_body,
        grid=(N // 1024,),
        in_specs=[
            pl.BlockSpec((1024, AUG), lambda i: (i, 0)),
            pl.BlockSpec((1, HID), lambda i: (0, 0)),
            pl.BlockSpec((HID, HID), lambda i: (0, 0)),
            pl.BlockSpec((HID, HID), lambda i: (0, 0)),
        ],
        out_specs=[
            pl.BlockSpec((1024, HID), lambda i: (i, 0)),
            pl.BlockSpec((1024, HID), lambda i: (i, 0)),
        ],
        out_shape=(
            jax.ShapeDtypeStruct((N, HID), jnp.float32),
            jax.ShapeDtypeStruct((N, HID), jnp.float32),
        ),
    )(numa, bias, w1, w2)


def _t3_body(q_ref, b_ref, w_ref, bb_ref, a_ref):
    ef = _gelu(q_ref[...] + b_ref[...])
    a_ref[...] = jnp.dot(ef, w_ref[...],
                         preferred_element_type=jnp.float32) + bb_ref[...]


def _t3(q, fcb, wgru, bgru):
    return pl.pallas_call(
        _t3_body,
        grid=(E // 2048,),
        in_specs=[
            pl.BlockSpec((2048, HID), lambda i: (i, 0)),
            pl.BlockSpec((1, HID), lambda i: (0, 0)),
            pl.BlockSpec((HID, 3 * OUT), lambda i: (0, 0)),
            pl.BlockSpec((1, 3 * OUT), lambda i: (0, 0)),
        ],
        out_specs=pl.BlockSpec((2048, 3 * OUT), lambda i: (i, 0)),
        out_shape=jax.ShapeDtypeStruct((E, 3 * OUT), jnp.float32),
    )(q, fcb, wgru, bgru)


def _t4_body(cnt_ref, a_ref, wz_ref, wr_ref, wh_ref, out_ref, h_s):
    i = pl.program_id(0)

    @pl.when(i == 0)
    def _():
        h_s[...] = jnp.zeros((G, OUT), jnp.float32)

    cnt = cnt_ref[...]
    cnt2 = cnt.reshape(G, 1)
    maxc = jnp.max(cnt)
    t0 = i * CH

    @pl.when(t0 < maxc)
    def _():
        wz = wz_ref[...]
        wr = wr_ref[...]
        wh = wh_ref[...]

        def body(t, h):
            row = a_ref[t]
            xz = row[:, :OUT]
            xr = row[:, OUT:2 * OUT]
            xh = row[:, 2 * OUT:]
            z = jax.nn.sigmoid(xz + jnp.dot(h, wz, preferred_element_type=jnp.float32))
            r = jax.nn.sigmoid(xr + jnp.dot(h, wr, preferred_element_type=jnp.float32))
            ht = _leaky(xh + jnp.dot(r * h, wh, preferred_element_type=jnp.float32), 0.01)
            nh = (1.0 - z) * h + z * ht
            valid = (t0 + t) < cnt2
            return jnp.where(valid, nh, h)

        h_s[...] = lax.fori_loop(0, CH, body, h_s[...])

    out_ref[...] = h_s[...]


def _t4(cnt, apad, wz, wr, wh):
    return pl.pallas_call(
        _t4_body,
        grid=(E // CH,),
        in_specs=[
            pl.BlockSpec((1, G), lambda i: (0, 0)),
            pl.BlockSpec((CH, G, 3 * OUT), lambda i: (i, 0, 0)),
            pl.BlockSpec((OUT, OUT), lambda i: (0, 0)),
            pl.BlockSpec((OUT, OUT), lambda i: (0, 0)),
            pl.BlockSpec((OUT, OUT), lambda i: (0, 0)),
        ],
        out_specs=pl.BlockSpec((G, OUT), lambda i: (0, 0)),
        out_shape=jax.ShapeDtypeStruct((G, OUT), jnp.float32),
        scratch_shapes=[pltpu.VMEM((G, OUT), jnp.float32)],
    )(cnt, apad, wz, wr, wh)


def kernel(x, edge_index, edge_attr, batch, emb, W1, asrc1, adst1, b1,
           W2, asrc2, adst2, b2, W3, asrc3, adst3, b3, fc_w, fc_b,
           wz_w, wz_b, wr_w, wr_b, wh_w, wh_b):
    src = edge_index[0]
    dst = edge_index[1]
    nidx = x[:, -1].astype(jnp.int32)
    x256 = x[:, :HID]

    embp = jnp.concatenate([emb, jnp.zeros((emb.shape[0], 128 - EMB), jnp.float32)], axis=1)
    embr, bsrc = _s1(nidx, src, batch, embp)
    pos, posd, cnt, off, boff = _t1(bsrc.reshape(E, 1), dst.reshape(E, 1))
    pos = pos.reshape(E)
    pairs = _s1b(src, dst, posd.reshape(E))
    boff = boff.reshape(128)

    hwa, es, ed, m = _t2a(x256, embr, W1[:HID], W1[HID:],
                          asrc1.reshape(HID, 1), adst1.reshape(HID, 1))
    numa = _bk(hwa, es.reshape(N), ed.reshape(N), m, pairs, boff)

    hwa, es, ed, m = _t2b(numa, W2, b1.reshape(1, HID),
                          asrc2.reshape(HID, 1), adst2.reshape(HID, 1))
    numa = _bk(hwa, es.reshape(N), ed.reshape(N), m, pairs, boff)

    hwa, es, ed, m = _t2b(numa, W3, b2.reshape(1, HID),
                          asrc3.reshape(HID, 1), adst3.reshape(HID, 1))
    numa = _bk(hwa, es.reshape(N), ed.reshape(N), m, pairs, boff)

    p1, p2 = _t5(numa, b3.reshape(1, HID), fc_w[:HID], fc_w[HID:])
    q = _s2(p1, p2, src, dst, pos)

    wgru = jnp.concatenate([wz_w[:HID], wr_w[:HID], wh_w[:HID]], axis=1)
    bgru = jnp.concatenate([wz_b, wr_b, wh_b]).reshape(1, 3 * OUT)
    a = _t3(q, fc_b.reshape(1, HID), wgru, bgru)

    apad = _s3(a, off.reshape(G))
    return _t4(cnt, apad.reshape(E, G, 3 * OUT),
               wz_w[HID:], wr_w[HID:], wh_w[HID:])
